# baseline (speedup 1.0000x reference)
import functools

import jax
import jax.numpy as jnp
import numpy as np
from jax.experimental import pallas as pl
from jax.experimental.pallas import tpu as pltpu

N_TOK = 121
SLAB = 128
PAD = 16
NPADDED = 160
WIN = 136
D = 64
NF = 9
NS = 16
BN_DEN = np.float32(np.sqrt(1.0 + 1e-5))
INV_DEG = np.float32(1.0) / (np.float32(8.0) + np.float32(1e-6))
F32 = jnp.float32


def _conv(xpad, wcol, bias, cout):
    cin = xpad.shape[-1]
    wnd = {dy: xpad[:, 12 + 11 * dy:12 + 11 * dy + WIN, :]
           .reshape(NS * WIN, cin) for dy in (-1, 0, 1)}
    accs = {-1: None, 0: None, 1: None}
    for k in range(9):
        dy, dx = k // 3 - 1, k % 3 - 1
        d = jnp.dot(wnd[dy], wcol[k], preferred_element_type=F32)
        accs[dx] = d if accs[dx] is None else accs[dx] + d
    pr = jax.lax.broadcasted_iota(jnp.int32, (NS * SLAB, 1), 0) % SLAB
    live = pr < N_TOK
    w_idx = pr % 11
    masks = {
        0: live.astype(F32),
        1: ((w_idx <= 9) & live).astype(F32),
        -1: ((w_idx >= 1) & live).astype(F32),
    }
    acc = None
    for dx in (-1, 0, 1):
        z = accs[dx].reshape(NS, WIN, cout)[:, 4 + dx:4 + dx + SLAB, :]
        zm = z.reshape(NS * SLAB, cout) * masks[dx]
        acc = zm if acc is None else acc + zm
    h = jax.nn.relu((acc + bias) / BN_DEN)
    return h * live.astype(F32)


def _repad(h, cout):
    return jnp.pad(h.reshape(NS, SLAB, cout), ((0, 0), (PAD, PAD), (0, 0)))


def _fg_kernel(xref, w1ref, b1ref, w2ref, b2ref, w3ref, b3ref,
               wsref, wnref, gbref, karef, kbref, ceref, garef,
               aw1ref, ab1ref, aw2ref, ab2ref, outref):
    xt = jnp.transpose(xref[...], (0, 2, 1))
    xtp = jnp.pad(xt, ((0, 0), (PAD, NPADDED - N_TOK - PAD), (0, 0)))
    h1 = _conv(xtp, w1ref[...], b1ref[...], 32)
    h2 = _conv(_repad(h1, 32), w2ref[...], b2ref[...], D)
    t = _conv(_repad(h2, D), w3ref[...], b3ref[...], D)
    t3 = t.reshape(NS, SLAB, D)

    xy = jax.lax.dot_general(t3, t3, (((2,), (2,)), ((0,), (0,))),
                             preferred_element_type=F32)
    xx = jnp.sum(t3 * t3, axis=2, keepdims=True)
    jj = jax.lax.broadcasted_iota(jnp.int32, (1, SLAB, SLAB), 2)
    xx_row = jnp.transpose(xx, (0, 2, 1))
    dist = jnp.clip(xx - 2.0 * xy + xx_row, 0.0, None)
    dist = jnp.where(jj >= N_TOK, jnp.inf, dist)

    cur = dist
    adj = jnp.zeros((NS, SLAB, SLAB), dtype=F32)
    for it in range(9):
        m = jnp.min(cur, axis=2, keepdims=True)
        hit = cur == m
        if it > 0:
            adj = adj + hit.astype(F32)
        cur = jnp.where(hit, jnp.inf, cur)

    hs = jnp.dot(t, wsref[...], preferred_element_type=F32)
    tw = jnp.dot(t, wnref[...], preferred_element_type=F32)
    hn3 = jax.lax.dot_general(adj, tw.reshape(NS, SLAB, D),
                              (((2,), (1,)), ((0,), (0,))),
                              preferred_element_type=F32)
    hn = hn3.reshape(NS * SLAB, D) * INV_DEG + tw
    gp = hs + hn + gbref[...]
    gp2 = gp * gp
    ka, kb = karef[...], kbref[...]
    num = ka[0:1] + ka[1:2] * gp + ka[2:3] * gp2
    den = 1.0 + jnp.abs(kb[0:1] * gp + kb[1:2] * gp2)
    g = (num / (den + 1e-8)) / BN_DEN

    ce, ga = ceref[...], garef[...]
    acc = jnp.zeros((NS * SLAB, D), dtype=F32)
    for j in range(NF):
        d = g - ce[j:j + 1]
        acc = acc + jnp.exp(-jnp.abs(ga[j:j + 1]) * (d * d))
    rbf = acc + np.float32(NF * 1e-10)
    a1 = jax.nn.relu(jnp.dot(rbf, aw1ref[...],
                             preferred_element_type=F32) + ab1ref[...])
    attn = jax.nn.sigmoid(jnp.dot(a1, aw2ref[...],
                                  preferred_element_type=F32) + ab2ref[...])
    att = ((g * attn + g) / BN_DEN).reshape(NS, SLAB, D)
    live = (jax.lax.broadcasted_iota(jnp.int32, (1, SLAB, 1), 1) < N_TOK)
    s = jnp.sum(jnp.where(live, att, 0.0), axis=1) / np.float32(N_TOK)
    mx = jnp.max(jnp.where(live, att, -jnp.inf), axis=1)
    outref[:, 0, :] = s + mx


def _head_kernel(pref, w1ref, b1ref, w2ref, b2ref, outref):
    z = jax.nn.relu((jnp.dot(pref[...], w1ref[...],
                             preferred_element_type=F32)
                     + b1ref[...]) / BN_DEN)
    outref[...] = (jnp.dot(z, w2ref[...], preferred_element_type=F32)
                   + b2ref[...])


def kernel(x, conv1_w, conv1_b, conv2_w, conv2_b, conv3_w, conv3_b,
           w_self, w_nb, gkan_bias, kan_a, kan_b, centers, gamma,
           attn_w1, attn_b1, attn_w2, attn_b2, fc1_w, fc1_b, fc2_w, fc2_b):
    B = x.shape[0]
    xn = x.reshape(B, x.shape[1], N_TOK)

    w1 = jnp.transpose(conv1_w, (2, 3, 1, 0)).reshape(9, 200, 32)
    w2 = jnp.transpose(conv2_w, (2, 3, 1, 0)).reshape(9, 32, 64)
    w3 = jnp.transpose(conv3_w, (2, 3, 1, 0)).reshape(9, 64, 64)

    const = lambda shape: pl.BlockSpec(shape, lambda i: (0,) * len(shape))
    pooled = pl.pallas_call(
        _fg_kernel,
        grid=(B // NS,),
        in_specs=[
            pl.BlockSpec((NS, 200, N_TOK), lambda i: (i, 0, 0)),
            const((9, 200, 32)), const((1, 32)),
            const((9, 32, 64)), const((1, 64)),
            const((9, 64, 64)), const((1, 64)),
            const((D, D)), const((D, D)), const((1, D)),
            const((3, D)), const((2, D)), const((NF, D)), const((NF, D)),
            const((D, 16)), const((1, 16)), const((16, D)), const((1, D)),
        ],
        out_specs=pl.BlockSpec((NS, 1, D), lambda i: (i, 0, 0)),
        out_shape=jax.ShapeDtypeStruct((B, 1, D), F32),
        compiler_params=pltpu.CompilerParams(
            dimension_semantics=("parallel",)),
    )(
        xn, w1, conv1_b[None, :], w2, conv2_b[None, :], w3,
        conv3_b[None, :], w_self, w_nb, gkan_bias[None, :],
        kan_a.T, kan_b.T, centers.T, gamma.T,
        attn_w1.T, attn_b1[None, :], attn_w2.T, attn_b2[None, :],
    )

    pooled = pooled.reshape(B, D)
    logits = pl.pallas_call(
        _head_kernel,
        in_specs=[
            pl.BlockSpec((B, D), lambda: (0, 0)),
            pl.BlockSpec((D, 128), lambda: (0, 0)),
            pl.BlockSpec((1, 128), lambda: (0, 0)),
            pl.BlockSpec((128, 16), lambda: (0, 0)),
            pl.BlockSpec((1, 16), lambda: (0, 0)),
        ],
        out_specs=pl.BlockSpec((B, 16), lambda: (0, 0)),
        out_shape=jax.ShapeDtypeStruct((B, 16), F32),
    )(pooled, fc1_w.T, fc1_b[None, :], fc2_w.T, fc2_b[None, :])
    return logits


# speedup vs baseline: 1.0197x; 1.0197x over previous
import functools

import jax
import jax.numpy as jnp
import numpy as np
from jax.experimental import pallas as pl
from jax.experimental.pallas import tpu as pltpu

N_TOK = 121
SLAB = 128
PAD = 16
NPADDED = 160
WIN = 136
D = 64
NF = 9
NS = 16
BN_DEN = np.float32(np.sqrt(1.0 + 1e-5))
INV_DEG = np.float32(1.0) / (np.float32(8.0) + np.float32(1e-6))
F32 = jnp.float32


def _conv(xpad, wcol, bias, cout):
    cin = xpad.shape[-1]
    accs = {-1: None, 0: None, 1: None}
    for k in range(9):
        dy, dx = k // 3 - 1, k % 3 - 1
        s = 11 * dy + dx
        sl = xpad[:, PAD + s:PAD + s + SLAB, :]
        d = jnp.dot(sl.reshape(NS * SLAB, cin), wcol[k],
                    preferred_element_type=F32)
        accs[dx] = d if accs[dx] is None else accs[dx] + d
    pr = jax.lax.broadcasted_iota(jnp.int32, (NS * SLAB, 1), 0) % SLAB
    live = pr < N_TOK
    w_idx = pr % 11
    acc = (accs[0] * live.astype(F32)
           + accs[1] * ((w_idx <= 9) & live).astype(F32)
           + accs[-1] * ((w_idx >= 1) & live).astype(F32))
    h = jax.nn.relu((acc + bias) / BN_DEN)
    return h * live.astype(F32)


def _repad(h, cout):
    return jnp.pad(h.reshape(NS, SLAB, cout), ((0, 0), (PAD, PAD), (0, 0)))


def _fg_kernel(xref, w1ref, b1ref, w2ref, b2ref, w3ref, b3ref,
               wsref, wnref, gbref, karef, kbref, ceref, garef,
               aw1ref, ab1ref, aw2ref, ab2ref, outref):
    xt = jnp.transpose(xref[...], (0, 2, 1))
    xtp = jnp.pad(xt, ((0, 0), (PAD, NPADDED - N_TOK - PAD), (0, 0)))
    h1 = _conv(xtp, w1ref[...], b1ref[...], 32)
    h2 = _conv(_repad(h1, 32), w2ref[...], b2ref[...], D)
    t = _conv(_repad(h2, D), w3ref[...], b3ref[...], D)
    t3 = t.reshape(NS, SLAB, D)

    xy = jax.lax.dot_general(t3, t3, (((2,), (2,)), ((0,), (0,))),
                             preferred_element_type=F32)
    xx = jnp.sum(t3 * t3, axis=2, keepdims=True)
    jj = jax.lax.broadcasted_iota(jnp.int32, (1, SLAB, SLAB), 2)
    xx_row = jnp.transpose(xx, (0, 2, 1))
    dist = jnp.clip(xx - 2.0 * xy + xx_row, 0.0, None)
    dist = jnp.where(jj >= N_TOK, jnp.inf, dist)

    cur = dist
    adj = jnp.zeros((NS, SLAB, SLAB), dtype=F32)
    for it in range(9):
        m = jnp.min(cur, axis=2, keepdims=True)
        hit = cur == m
        if it > 0:
            adj = adj + hit.astype(F32)
        cur = jnp.where(hit, jnp.inf, cur)

    hs = jnp.dot(t, wsref[...], preferred_element_type=F32)
    tw = jnp.dot(t, wnref[...], preferred_element_type=F32)
    hn3 = jax.lax.dot_general(adj, tw.reshape(NS, SLAB, D),
                              (((2,), (1,)), ((0,), (0,))),
                              preferred_element_type=F32)
    hn = hn3.reshape(NS * SLAB, D) * INV_DEG + tw
    gp = hs + hn + gbref[...]
    gp2 = gp * gp
    ka, kb = karef[...], kbref[...]
    num = ka[0:1] + ka[1:2] * gp + ka[2:3] * gp2
    den = 1.0 + jnp.abs(kb[0:1] * gp + kb[1:2] * gp2)
    g = (num / (den + 1e-8)) / BN_DEN

    ce, ga = ceref[...], garef[...]
    acc = jnp.zeros((NS * SLAB, D), dtype=F32)
    for j in range(NF):
        d = g - ce[j:j + 1]
        acc = acc + jnp.exp(-jnp.abs(ga[j:j + 1]) * (d * d))
    rbf = acc + np.float32(NF * 1e-10)
    a1 = jax.nn.relu(jnp.dot(rbf, aw1ref[...],
                             preferred_element_type=F32) + ab1ref[...])
    attn = jax.nn.sigmoid(jnp.dot(a1, aw2ref[...],
                                  preferred_element_type=F32) + ab2ref[...])
    att = ((g * attn + g) / BN_DEN).reshape(NS, SLAB, D)
    live = (jax.lax.broadcasted_iota(jnp.int32, (1, SLAB, 1), 1) < N_TOK)
    s = jnp.sum(jnp.where(live, att, 0.0), axis=1) / np.float32(N_TOK)
    mx = jnp.max(jnp.where(live, att, -jnp.inf), axis=1)
    outref[:, 0, :] = s + mx


def _head_kernel(pref, w1ref, b1ref, w2ref, b2ref, outref):
    z = jax.nn.relu((jnp.dot(pref[...], w1ref[...],
                             preferred_element_type=F32)
                     + b1ref[...]) / BN_DEN)
    outref[...] = (jnp.dot(z, w2ref[...], preferred_element_type=F32)
                   + b2ref[...])


def kernel(x, conv1_w, conv1_b, conv2_w, conv2_b, conv3_w, conv3_b,
           w_self, w_nb, gkan_bias, kan_a, kan_b, centers, gamma,
           attn_w1, attn_b1, attn_w2, attn_b2, fc1_w, fc1_b, fc2_w, fc2_b):
    B = x.shape[0]
    xn = x.reshape(B, x.shape[1], N_TOK)

    w1 = jnp.transpose(conv1_w, (2, 3, 1, 0)).reshape(9, 200, 32)
    w2 = jnp.transpose(conv2_w, (2, 3, 1, 0)).reshape(9, 32, 64)
    w3 = jnp.transpose(conv3_w, (2, 3, 1, 0)).reshape(9, 64, 64)

    const = lambda shape: pl.BlockSpec(shape, lambda i: (0,) * len(shape))
    pooled = pl.pallas_call(
        _fg_kernel,
        grid=(B // NS,),
        in_specs=[
            pl.BlockSpec((NS, 200, N_TOK), lambda i: (i, 0, 0)),
            const((9, 200, 32)), const((1, 32)),
            const((9, 32, 64)), const((1, 64)),
            const((9, 64, 64)), const((1, 64)),
            const((D, D)), const((D, D)), const((1, D)),
            const((3, D)), const((2, D)), const((NF, D)), const((NF, D)),
            const((D, 16)), const((1, 16)), const((16, D)), const((1, D)),
        ],
        out_specs=pl.BlockSpec((NS, 1, D), lambda i: (i, 0, 0)),
        out_shape=jax.ShapeDtypeStruct((B, 1, D), F32),
        compiler_params=pltpu.CompilerParams(
            dimension_semantics=("parallel",)),
    )(
        xn, w1, conv1_b[None, :], w2, conv2_b[None, :], w3,
        conv3_b[None, :], w_self, w_nb, gkan_bias[None, :],
        kan_a.T, kan_b.T, centers.T, gamma.T,
        attn_w1.T, attn_b1[None, :], attn_w2.T, attn_b2[None, :],
    )

    pooled = pooled.reshape(B, D)
    logits = pl.pallas_call(
        _head_kernel,
        in_specs=[
            pl.BlockSpec((B, D), lambda: (0, 0)),
            pl.BlockSpec((D, 128), lambda: (0, 0)),
            pl.BlockSpec((1, 128), lambda: (0, 0)),
            pl.BlockSpec((128, 16), lambda: (0, 0)),
            pl.BlockSpec((1, 16), lambda: (0, 0)),
        ],
        out_specs=pl.BlockSpec((B, 16), lambda: (0, 0)),
        out_shape=jax.ShapeDtypeStruct((B, 16), F32),
    )(pooled, fc1_w.T, fc1_b[None, :], fc2_w.T, fc2_b[None, :])
    return logits


# speedup vs baseline: 1.0345x; 1.0146x over previous
import functools

import jax
import jax.numpy as jnp
import numpy as np
from jax.experimental import pallas as pl
from jax.experimental.pallas import tpu as pltpu

N_TOK = 121
SLAB = 128
PAD = 16
NPADDED = 160
WIN = 136
D = 64
NF = 9
NS = 32
BN_DEN = np.float32(np.sqrt(1.0 + 1e-5))
INV_DEG = np.float32(1.0) / (np.float32(8.0) + np.float32(1e-6))
F32 = jnp.float32


def _conv(xpad, wcol, bias, cout):
    cin = xpad.shape[-1]
    accs = {-1: None, 0: None, 1: None}
    for k in range(9):
        dy, dx = k // 3 - 1, k % 3 - 1
        s = 11 * dy + dx
        sl = xpad[:, PAD + s:PAD + s + SLAB, :]
        d = jnp.dot(sl.reshape(NS * SLAB, cin), wcol[k],
                    preferred_element_type=F32)
        accs[dx] = d if accs[dx] is None else accs[dx] + d
    pr = jax.lax.broadcasted_iota(jnp.int32, (NS * SLAB, 1), 0) % SLAB
    live = pr < N_TOK
    w_idx = pr % 11
    acc = (accs[0] * live.astype(F32)
           + accs[1] * ((w_idx <= 9) & live).astype(F32)
           + accs[-1] * ((w_idx >= 1) & live).astype(F32))
    h = jax.nn.relu((acc + bias) / BN_DEN)
    return h * live.astype(F32)


def _repad(h, cout):
    return jnp.pad(h.reshape(NS, SLAB, cout), ((0, 0), (PAD, PAD), (0, 0)))


def _fg_kernel(xref, w1ref, b1ref, w2ref, b2ref, w3ref, b3ref,
               wsref, wnref, gbref, karef, kbref, ceref, garef,
               aw1ref, ab1ref, aw2ref, ab2ref, outref):
    xt = jnp.transpose(xref[...], (0, 2, 1))
    xtp = jnp.pad(xt, ((0, 0), (PAD, NPADDED - N_TOK - PAD), (0, 0)))
    h1 = _conv(xtp, w1ref[...], b1ref[...], 32)
    h2 = _conv(_repad(h1, 32), w2ref[...], b2ref[...], D)
    t = _conv(_repad(h2, D), w3ref[...], b3ref[...], D)
    t3 = t.reshape(NS, SLAB, D)

    xy = jax.lax.dot_general(t3, t3, (((2,), (2,)), ((0,), (0,))),
                             preferred_element_type=F32)
    xx = jnp.sum(t3 * t3, axis=2, keepdims=True)
    jj = jax.lax.broadcasted_iota(jnp.int32, (1, SLAB, SLAB), 2)
    xx_row = jnp.transpose(xx, (0, 2, 1))
    dist = jnp.clip(xx - 2.0 * xy + xx_row, 0.0, None)
    dist = jnp.where(jj >= N_TOK, jnp.inf, dist)

    cur = dist
    adj = jnp.zeros((NS, SLAB, SLAB), dtype=F32)
    for it in range(9):
        m = jnp.min(cur, axis=2, keepdims=True)
        hit = cur == m
        if it > 0:
            adj = adj + hit.astype(F32)
        cur = jnp.where(hit, jnp.inf, cur)

    hs = jnp.dot(t, wsref[...], preferred_element_type=F32)
    tw = jnp.dot(t, wnref[...], preferred_element_type=F32)
    hn3 = jax.lax.dot_general(adj, tw.reshape(NS, SLAB, D),
                              (((2,), (1,)), ((0,), (0,))),
                              preferred_element_type=F32)
    hn = hn3.reshape(NS * SLAB, D) * INV_DEG + tw
    gp = hs + hn + gbref[...]
    gp2 = gp * gp
    ka, kb = karef[...], kbref[...]
    num = ka[0:1] + ka[1:2] * gp + ka[2:3] * gp2
    den = 1.0 + jnp.abs(kb[0:1] * gp + kb[1:2] * gp2)
    g = (num / (den + 1e-8)) / BN_DEN

    ce, ga = ceref[...], garef[...]
    acc = jnp.zeros((NS * SLAB, D), dtype=F32)
    for j in range(NF):
        d = g - ce[j:j + 1]
        acc = acc + jnp.exp(-jnp.abs(ga[j:j + 1]) * (d * d))
    rbf = acc + np.float32(NF * 1e-10)
    a1 = jax.nn.relu(jnp.dot(rbf, aw1ref[...],
                             preferred_element_type=F32) + ab1ref[...])
    attn = jax.nn.sigmoid(jnp.dot(a1, aw2ref[...],
                                  preferred_element_type=F32) + ab2ref[...])
    att = ((g * attn + g) / BN_DEN).reshape(NS, SLAB, D)
    live = (jax.lax.broadcasted_iota(jnp.int32, (1, SLAB, 1), 1) < N_TOK)
    s = jnp.sum(jnp.where(live, att, 0.0), axis=1) / np.float32(N_TOK)
    mx = jnp.max(jnp.where(live, att, -jnp.inf), axis=1)
    outref[:, 0, :] = s + mx


def _head_kernel(pref, w1ref, b1ref, w2ref, b2ref, outref):
    z = jax.nn.relu((jnp.dot(pref[...], w1ref[...],
                             preferred_element_type=F32)
                     + b1ref[...]) / BN_DEN)
    outref[...] = (jnp.dot(z, w2ref[...], preferred_element_type=F32)
                   + b2ref[...])


def kernel(x, conv1_w, conv1_b, conv2_w, conv2_b, conv3_w, conv3_b,
           w_self, w_nb, gkan_bias, kan_a, kan_b, centers, gamma,
           attn_w1, attn_b1, attn_w2, attn_b2, fc1_w, fc1_b, fc2_w, fc2_b):
    B = x.shape[0]
    xn = x.reshape(B, x.shape[1], N_TOK)

    w1 = jnp.transpose(conv1_w, (2, 3, 1, 0)).reshape(9, 200, 32)
    w2 = jnp.transpose(conv2_w, (2, 3, 1, 0)).reshape(9, 32, 64)
    w3 = jnp.transpose(conv3_w, (2, 3, 1, 0)).reshape(9, 64, 64)

    const = lambda shape: pl.BlockSpec(shape, lambda i: (0,) * len(shape))
    pooled = pl.pallas_call(
        _fg_kernel,
        grid=(B // NS,),
        in_specs=[
            pl.BlockSpec((NS, 200, N_TOK), lambda i: (i, 0, 0)),
            const((9, 200, 32)), const((1, 32)),
            const((9, 32, 64)), const((1, 64)),
            const((9, 64, 64)), const((1, 64)),
            const((D, D)), const((D, D)), const((1, D)),
            const((3, D)), const((2, D)), const((NF, D)), const((NF, D)),
            const((D, 16)), const((1, 16)), const((16, D)), const((1, D)),
        ],
        out_specs=pl.BlockSpec((NS, 1, D), lambda i: (i, 0, 0)),
        out_shape=jax.ShapeDtypeStruct((B, 1, D), F32),
        compiler_params=pltpu.CompilerParams(
            dimension_semantics=("parallel",)),
    )(
        xn, w1, conv1_b[None, :], w2, conv2_b[None, :], w3,
        conv3_b[None, :], w_self, w_nb, gkan_bias[None, :],
        kan_a.T, kan_b.T, centers.T, gamma.T,
        attn_w1.T, attn_b1[None, :], attn_w2.T, attn_b2[None, :],
    )

    pooled = pooled.reshape(B, D)
    logits = pl.pallas_call(
        _head_kernel,
        in_specs=[
            pl.BlockSpec((B, D), lambda: (0, 0)),
            pl.BlockSpec((D, 128), lambda: (0, 0)),
            pl.BlockSpec((1, 128), lambda: (0, 0)),
            pl.BlockSpec((128, 16), lambda: (0, 0)),
            pl.BlockSpec((1, 16), lambda: (0, 0)),
        ],
        out_specs=pl.BlockSpec((B, 16), lambda: (0, 0)),
        out_shape=jax.ShapeDtypeStruct((B, 16), F32),
    )(pooled, fc1_w.T, fc1_b[None, :], fc2_w.T, fc2_b[None, :])
    return logits
